# baseline (speedup 1.0000x reference)
import functools

import jax
import jax.numpy as jnp
from jax import lax
from jax.experimental import pallas as pl
from jax.experimental.pallas import tpu as pltpu
from jax.experimental.pallas import tpu_sc as plsc

_T = 1000
_D = 128
_B = 16384
_NC = 2
_NS = 16
_NW = _NC * _NS
_BPW = _B // _NW
_CHUNKS = [128, 128, 128, 128]
_NCHUNK = len(_CHUNKS)
_OFFS = [sum(_CHUNKS[:j]) for j in range(_NCHUNK)]
_HBM_CHUNKS = 1
_STAGE_SPLIT = [(k * 128, 128) for k in range(7)] + [(896, 104)]

_mesh = plsc.VectorSubcoreMesh(core_axis_name="c", subcore_axis_name="s")


@functools.partial(
    pl.kernel,
    mesh=_mesh,
    out_type=jax.ShapeDtypeStruct((_B, _D), jnp.float32),
    scratch_types=[
        pltpu.VMEM((_BPW,), jnp.int32),
        pltpu.VMEM((_BPW, _D), jnp.float32),
        pltpu.VMEM_SHARED((_T, _D), jnp.float32),
    ]
    + [pltpu.SemaphoreType.DMA] * (_NCHUNK + 1),
)
def _lookup(te_hbm, t_hbm, out_hbm, idx_v, rows_v, table_s, *sems):
    gsems, wsem = sems[:_NCHUNK], sems[_NCHUNK]
    sid = lax.axis_index("s")
    wid = sid * _NC + lax.axis_index("c")
    base = wid * _BPW

    pltpu.sync_copy(t_hbm.at[pl.ds(base, _BPW)], idx_v)

    gathers = []
    for j in range(_HBM_CHUNKS):
        gathers.append(
            pltpu.async_copy(
                te_hbm.at[idx_v.at[pl.ds(_OFFS[j], _CHUNKS[j])]],
                rows_v.at[pl.ds(_OFFS[j], _CHUNKS[j])],
                gsems[j],
            )
        )

    for k, (r0, nrows) in enumerate(_STAGE_SPLIT):
        @pl.when(sid == k)
        def _(r0=r0, nrows=nrows):
            pltpu.sync_copy(
                te_hbm.at[pl.ds(r0, nrows)],
                table_s.at[pl.ds(r0, nrows)],
            )
    plsc.subcore_barrier()

    for j in range(_HBM_CHUNKS, _NCHUNK):
        gathers.append(
            pltpu.async_copy(
                table_s.at[idx_v.at[pl.ds(_OFFS[j], _CHUNKS[j])]],
                rows_v.at[pl.ds(_OFFS[j], _CHUNKS[j])],
                gsems[j],
            )
        )

    writes = []
    for j in range(_NCHUNK):
        gathers[j].wait()
        writes.append(
            pltpu.async_copy(
                rows_v.at[pl.ds(_OFFS[j], _CHUNKS[j])],
                out_hbm.at[pl.ds(base + _OFFS[j], _CHUNKS[j])],
                wsem,
            )
        )
    for c in writes:
        c.wait()


def kernel(te, t):
    if t.dtype != jnp.int32:
        t = t.astype(jnp.int32)
    return _lookup(te, t)


# speedup vs baseline: 1.0117x; 1.0117x over previous
import functools

import jax
import jax.numpy as jnp
from jax import lax
from jax.experimental import pallas as pl
from jax.experimental.pallas import tpu as pltpu
from jax.experimental.pallas import tpu_sc as plsc

_T = 1000
_D = 128
_B = 16384
_NC = 2
_NS = 16
_NW = _NC * _NS
_BPW = _B // _NW
_CHUNK = 128
_NCHUNK = _BPW // _CHUNK
_STAGE_SPLIT = [(k * 128, 128) for k in range(7)] + [(896, 104)]

_mesh = plsc.VectorSubcoreMesh(core_axis_name="c", subcore_axis_name="s")


@functools.partial(
    pl.kernel,
    mesh=_mesh,
    out_type=jax.ShapeDtypeStruct((_B, _D), jnp.float32),
    scratch_types=[
        pltpu.VMEM((_BPW,), jnp.int32),
        pltpu.VMEM((_BPW, _D), jnp.float32),
        pltpu.VMEM_SHARED((_T, _D), jnp.float32),
    ]
    + [pltpu.SemaphoreType.DMA] * (_NCHUNK + 1),
)
def _lookup(te_hbm, t_hbm, out_hbm, idx_v, rows_v, table_s, *sems):
    gsems, wsem = sems[:_NCHUNK], sems[_NCHUNK]
    sid = lax.axis_index("s")
    wid = sid * _NC + lax.axis_index("c")
    base = wid * _BPW

    pltpu.sync_copy(t_hbm.at[pl.ds(base, _BPW)], idx_v)

    for k, (r0, nrows) in enumerate(_STAGE_SPLIT):
        @pl.when(sid == k)
        def _(r0=r0, nrows=nrows):
            pltpu.sync_copy(
                te_hbm.at[pl.ds(r0, nrows)],
                table_s.at[pl.ds(r0, nrows)],
            )
    plsc.subcore_barrier()

    gathers = []
    for j in range(_NCHUNK):
        gathers.append(
            pltpu.async_copy(
                table_s.at[idx_v.at[pl.ds(j * _CHUNK, _CHUNK)]],
                rows_v.at[pl.ds(j * _CHUNK, _CHUNK)],
                gsems[j],
            )
        )
    writes = []
    for j in range(_NCHUNK):
        gathers[j].wait()
        writes.append(
            pltpu.async_copy(
                rows_v.at[pl.ds(j * _CHUNK, _CHUNK)],
                out_hbm.at[pl.ds(base + j * _CHUNK, _CHUNK)],
                wsem,
            )
        )
    for c in writes:
        c.wait()


def kernel(te, t):
    if t.dtype != jnp.int32:
        t = t.astype(jnp.int32)
    return _lookup(te, t)


# speedup vs baseline: 1.0302x; 1.0182x over previous
import functools

import jax
import jax.numpy as jnp
from jax import lax
from jax.experimental import pallas as pl
from jax.experimental.pallas import tpu as pltpu
from jax.experimental.pallas import tpu_sc as plsc

_T = 1000
_D = 128
_B = 16384
_NC = 2
_NS = 16
_NW = _NC * _NS
_BPW = _B // _NW
_CHUNK = 128
_NCHUNK = _BPW // _CHUNK
_STAGE_SPLIT = [(k * 128, 128) for k in range(7)] + [(896, 104)]

_mesh = plsc.VectorSubcoreMesh(core_axis_name="c", subcore_axis_name="s")


@functools.partial(
    pl.kernel,
    mesh=_mesh,
    out_type=jax.ShapeDtypeStruct((_B, _D), jnp.float32),
    scratch_types=[
        pltpu.VMEM((_BPW,), jnp.int32),
        pltpu.VMEM((_BPW, _D), jnp.float32),
        pltpu.VMEM_SHARED((_T, _D), jnp.float32),
    ]
    + [pltpu.SemaphoreType.DMA] * (_NCHUNK + 2),
)
def _lookup(te_hbm, t_hbm, out_hbm, idx_v, rows_v, table_s, *sems):
    gsems, wsem, isem = sems[:_NCHUNK], sems[_NCHUNK], sems[_NCHUNK + 1]
    sid = lax.axis_index("s")
    wid = sid * _NC + lax.axis_index("c")
    base = wid * _BPW

    idx_cp = pltpu.async_copy(t_hbm.at[pl.ds(base, _BPW)], idx_v, isem)

    for k, (r0, nrows) in enumerate(_STAGE_SPLIT):
        @pl.when(sid == k)
        def _(r0=r0, nrows=nrows):
            pltpu.sync_copy(
                te_hbm.at[pl.ds(r0, nrows)],
                table_s.at[pl.ds(r0, nrows)],
            )
    idx_cp.wait()
    plsc.subcore_barrier()

    gathers = []
    for j in range(_NCHUNK):
        gathers.append(
            pltpu.async_copy(
                table_s.at[idx_v.at[pl.ds(j * _CHUNK, _CHUNK)]],
                rows_v.at[pl.ds(j * _CHUNK, _CHUNK)],
                gsems[j],
            )
        )
    writes = []
    for j in range(_NCHUNK):
        gathers[j].wait()
        writes.append(
            pltpu.async_copy(
                rows_v.at[pl.ds(j * _CHUNK, _CHUNK)],
                out_hbm.at[pl.ds(base + j * _CHUNK, _CHUNK)],
                wsem,
            )
        )
    for c in writes:
        c.wait()


def kernel(te, t):
    if t.dtype != jnp.int32:
        t = t.astype(jnp.int32)
    return _lookup(te, t)


# speedup vs baseline: 1.0427x; 1.0122x over previous
import functools

import jax
import jax.numpy as jnp
from jax import lax
from jax.experimental import pallas as pl
from jax.experimental.pallas import tpu as pltpu
from jax.experimental.pallas import tpu_sc as plsc

_T = 1000
_D = 128
_B = 16384
_NC = 2
_NS = 16
_NW = _NC * _NS
_BPW = _B // _NW
_CHUNK = 64
_NCHUNK = _BPW // _CHUNK
_STAGE_SPLIT = [(k * 128, 128) for k in range(7)] + [(896, 104)]

_mesh = plsc.VectorSubcoreMesh(core_axis_name="c", subcore_axis_name="s")


@functools.partial(
    pl.kernel,
    mesh=_mesh,
    out_type=jax.ShapeDtypeStruct((_B, _D), jnp.float32),
    scratch_types=[
        pltpu.VMEM((_BPW,), jnp.int32),
        pltpu.VMEM((_BPW, _D), jnp.float32),
        pltpu.VMEM_SHARED((_T, _D), jnp.float32),
    ]
    + [pltpu.SemaphoreType.DMA] * (_NCHUNK + 2),
)
def _lookup(te_hbm, t_hbm, out_hbm, idx_v, rows_v, table_s, *sems):
    gsems, wsem, isem = sems[:_NCHUNK], sems[_NCHUNK], sems[_NCHUNK + 1]
    sid = lax.axis_index("s")
    wid = sid * _NC + lax.axis_index("c")
    base = wid * _BPW

    idx_cp = pltpu.async_copy(t_hbm.at[pl.ds(base, _BPW)], idx_v, isem)

    for k, (r0, nrows) in enumerate(_STAGE_SPLIT):
        @pl.when(sid == k)
        def _(r0=r0, nrows=nrows):
            pltpu.sync_copy(
                te_hbm.at[pl.ds(r0, nrows)],
                table_s.at[pl.ds(r0, nrows)],
            )
    idx_cp.wait()
    plsc.subcore_barrier()

    gathers = []
    for j in range(_NCHUNK):
        gathers.append(
            pltpu.async_copy(
                table_s.at[idx_v.at[pl.ds(j * _CHUNK, _CHUNK)]],
                rows_v.at[pl.ds(j * _CHUNK, _CHUNK)],
                gsems[j],
            )
        )
    writes = []
    for j in range(_NCHUNK):
        gathers[j].wait()
        writes.append(
            pltpu.async_copy(
                rows_v.at[pl.ds(j * _CHUNK, _CHUNK)],
                out_hbm.at[pl.ds(base + j * _CHUNK, _CHUNK)],
                wsem,
            )
        )
    for c in writes:
        c.wait()


def kernel(te, t):
    if t.dtype != jnp.int32:
        t = t.astype(jnp.int32)
    return _lookup(te, t)


# speedup vs baseline: 1.0476x; 1.0047x over previous
import functools

import jax
import jax.numpy as jnp
from jax import lax
from jax.experimental import pallas as pl
from jax.experimental.pallas import tpu as pltpu
from jax.experimental.pallas import tpu_sc as plsc

_T = 1000
_D = 128
_B = 16384
_NC = 2
_NS = 16
_NW = _NC * _NS
_BPW = _B // _NW
_CHUNK = 32
_NCHUNK = _BPW // _CHUNK
_STAGE_SPLIT = [(k * 128, 128) for k in range(7)] + [(896, 104)]

_mesh = plsc.VectorSubcoreMesh(core_axis_name="c", subcore_axis_name="s")


@functools.partial(
    pl.kernel,
    mesh=_mesh,
    out_type=jax.ShapeDtypeStruct((_B, _D), jnp.float32),
    scratch_types=[
        pltpu.VMEM((_BPW,), jnp.int32),
        pltpu.VMEM((_BPW, _D), jnp.float32),
        pltpu.VMEM_SHARED((_T, _D), jnp.float32),
    ]
    + [pltpu.SemaphoreType.DMA] * (_NCHUNK + 2),
)
def _lookup(te_hbm, t_hbm, out_hbm, idx_v, rows_v, table_s, *sems):
    gsems, wsem, isem = sems[:_NCHUNK], sems[_NCHUNK], sems[_NCHUNK + 1]
    sid = lax.axis_index("s")
    wid = sid * _NC + lax.axis_index("c")
    base = wid * _BPW

    idx_cp = pltpu.async_copy(t_hbm.at[pl.ds(base, _BPW)], idx_v, isem)

    for k, (r0, nrows) in enumerate(_STAGE_SPLIT):
        @pl.when(sid == k)
        def _(r0=r0, nrows=nrows):
            pltpu.sync_copy(
                te_hbm.at[pl.ds(r0, nrows)],
                table_s.at[pl.ds(r0, nrows)],
            )
    idx_cp.wait()
    plsc.subcore_barrier()

    gathers = []
    for j in range(_NCHUNK):
        gathers.append(
            pltpu.async_copy(
                table_s.at[idx_v.at[pl.ds(j * _CHUNK, _CHUNK)]],
                rows_v.at[pl.ds(j * _CHUNK, _CHUNK)],
                gsems[j],
            )
        )
    writes = []
    for j in range(_NCHUNK):
        gathers[j].wait()
        writes.append(
            pltpu.async_copy(
                rows_v.at[pl.ds(j * _CHUNK, _CHUNK)],
                out_hbm.at[pl.ds(base + j * _CHUNK, _CHUNK)],
                wsem,
            )
        )
    for c in writes:
        c.wait()


def kernel(te, t):
    if t.dtype != jnp.int32:
        t = t.astype(jnp.int32)
    return _lookup(te, t)
